# Initial kernel scaffold; baseline (speedup 1.0000x reference)
import jax
import jax.numpy as jnp
from jax.experimental import pallas as pl


def kernel(x, pos_table):
    raise NotImplementedError("write your pallas kernel here")


# speedup vs baseline: 5.0680x; 5.0680x over previous
import jax
import jax.numpy as jnp
from jax.experimental import pallas as pl


def kernel(x, pos_table):
    B, T, D = x.shape
    TILE = 512

    def body(tbl_ref, out_ref):
        out_ref[...] = jnp.broadcast_to(tbl_ref[...][None], (B, TILE, D))

    return pl.pallas_call(
        body,
        grid=(T // TILE,),
        in_specs=[pl.BlockSpec((TILE, D), lambda i: (i, 0))],
        out_specs=pl.BlockSpec((B, TILE, D), lambda i: (0, i, 0)),
        out_shape=jax.ShapeDtypeStruct((B, T, D), jnp.float32),
    )(pos_table)


# speedup vs baseline: 5.1733x; 1.0208x over previous
import jax
import jax.numpy as jnp
from jax.experimental import pallas as pl


def kernel(x, pos_table):
    B, T, D = x.shape
    TILE = 1024

    def body(tbl_ref, out_ref):
        out_ref[...] = jnp.broadcast_to(tbl_ref[...][None], (B, TILE, D))

    return pl.pallas_call(
        body,
        grid=(T // TILE,),
        in_specs=[pl.BlockSpec((TILE, D), lambda i: (i, 0))],
        out_specs=pl.BlockSpec((B, TILE, D), lambda i: (0, i, 0)),
        out_shape=jax.ShapeDtypeStruct((B, T, D), jnp.float32),
    )(pos_table)
